# baseline (speedup 1.0000x reference)
import functools

import jax
import jax.numpy as jnp
from jax import lax
from jax.experimental import pallas as pl
from jax.experimental.pallas import tpu as pltpu
from jax.experimental.pallas import tpu_sc as plsc

EPSILON = 1e-8
CUTOFF = 5.0
NODE_SIZE = 256
EDGE_SIZE = 20
N_NODES = 10000
N_EDGES = 160000

F_WIDTH = 896


_MLP_BLOCK = 1000


def _mlp_body(x_ref, w1_ref, b1_ref, w2_ref, b2_ref, o_ref):
    x = x_ref[...]
    h = jnp.dot(x, w1_ref[...], preferred_element_type=jnp.float32) + b1_ref[...]
    h = h * jax.nn.sigmoid(h)
    o_ref[...] = jnp.dot(h, w2_ref[...], preferred_element_type=jnp.float32) + b2_ref[...]


def _node_mlp(node_scalar, W1, b1, W2, b2):
    n_blocks = N_NODES // _MLP_BLOCK
    return pl.pallas_call(
        _mlp_body,
        grid=(n_blocks,),
        in_specs=[
            pl.BlockSpec((_MLP_BLOCK, NODE_SIZE), lambda i: (i, 0)),
            pl.BlockSpec((NODE_SIZE, NODE_SIZE), lambda i: (0, 0)),
            pl.BlockSpec((1, NODE_SIZE), lambda i: (0, 0)),
            pl.BlockSpec((NODE_SIZE, NODE_SIZE * 3), lambda i: (0, 0)),
            pl.BlockSpec((1, NODE_SIZE * 3), lambda i: (0, 0)),
        ],
        out_specs=pl.BlockSpec((_MLP_BLOCK, NODE_SIZE * 3), lambda i: (i, 0)),
        out_shape=jax.ShapeDtypeStruct((N_NODES, NODE_SIZE * 3), jnp.float32),
    )(node_scalar, W1, b1.reshape(1, -1), W2, b2.reshape(1, -1))


_EDGE_BLOCK = 2000


def _filter_body(dist_ref, diff_ref, wf_ref, bf_ref, o_ref):
    d = dist_ref[...]
    dm = jnp.maximum(d, EPSILON)
    freq = (lax.broadcasted_iota(jnp.int32, (1, EDGE_SIZE), 1).astype(jnp.float32)
            + 1.0) * (jnp.pi / CUTOFF)
    basis = jnp.sin(dm * freq) / dm
    fw = jnp.dot(basis, wf_ref[...], preferred_element_type=jnp.float32) + bf_ref[...]
    cut = jnp.where(d < CUTOFF, 0.5 * (jnp.cos(d * (jnp.pi / CUTOFF)) + 1.0), 0.0)
    fw = fw * cut
    dirpad = jnp.concatenate(
        [diff_ref[...] / dm, jnp.zeros((_EDGE_BLOCK, 125), jnp.float32)], axis=1)
    o_ref[...] = jnp.concatenate([fw, dirpad], axis=1)


def _edge_filter(edge_dist, edge_diff, Wf, bf):
    n_blocks = N_EDGES // _EDGE_BLOCK
    return pl.pallas_call(
        _filter_body,
        grid=(n_blocks,),
        in_specs=[
            pl.BlockSpec((_EDGE_BLOCK, 1), lambda i: (i, 0)),
            pl.BlockSpec((_EDGE_BLOCK, 3), lambda i: (i, 0)),
            pl.BlockSpec((EDGE_SIZE, NODE_SIZE * 3), lambda i: (0, 0)),
            pl.BlockSpec((1, NODE_SIZE * 3), lambda i: (0, 0)),
        ],
        out_specs=pl.BlockSpec((_EDGE_BLOCK, F_WIDTH), lambda i: (i, 0)),
        out_shape=jax.ShapeDtypeStruct((N_EDGES, F_WIDTH), jnp.float32),
    )(edge_dist.reshape(N_EDGES, 1), edge_diff, Wf, bf.reshape(1, -1))


BIN_SIZE = 500
N_BINS = 10
ACC_ROWS = BIN_SIZE + 8
E_PER_TILE = N_EDGES // 16
SEG = 2000
N_SEG = E_PER_TILE // SEG
SEG_VREG = SEG // 16
LIST_LEN = SEG + 32
K = 16


@functools.cache
def _build_sc_message(core):
    mesh = plsc.VectorSubcoreMesh(core_axis_name="c", subcore_axis_name="s",
                                  num_cores=1)
    return pl.kernel(
        functools.partial(_sc_message_body, core),
        mesh=mesh,
        out_type=(
            jax.ShapeDtypeStruct((N_NODES // 2, 1, NODE_SIZE), jnp.float32),
            jax.ShapeDtypeStruct((N_NODES // 2, 1, NODE_SIZE * 3), jnp.float32),
        ),
        scratch_types=[
            pltpu.VMEM((SEG,), jnp.int32),
            pltpu.VMEM((SEG,), jnp.int32),
            pltpu.VMEM((LIST_LEN,), jnp.int32),
            pltpu.VMEM((LIST_LEN,), jnp.int32),
            pltpu.VMEM((LIST_LEN,), jnp.int32),
            pltpu.VMEM((16,), jnp.int32),
            [pltpu.VMEM((K, 1, F_WIDTH), jnp.float32) for _ in range(2)],
            [pltpu.VMEM((K, 1, NODE_SIZE * 3), jnp.float32) for _ in range(2)],
            [pltpu.VMEM((K, 1, NODE_SIZE * 3), jnp.float32) for _ in range(2)],
            [pltpu.VMEM((K, 1, NODE_SIZE), jnp.float32) for _ in range(2)],
            pltpu.VMEM_SHARED((ACC_ROWS, 1, NODE_SIZE), jnp.float32),
            pltpu.VMEM_SHARED((ACC_ROWS, 1, NODE_SIZE * 3), jnp.float32),
            pltpu.SemaphoreType.DMA,
            pltpu.SemaphoreType.DMA,
        ],
        compiler_params=pltpu.CompilerParams(needs_layout_passes=False),
    )


def _sc_message_body(core, so_hbm, nv_hbm, f_hbm, edst_hbm, esrc_hbm, ns_hbm,
                     out_s, out_v,
                     dseg, sseg, list_eid, list_src, list_dstl, idx_dst,
                     frows2, srows2, vrows2, msgs2, acc_s, acc_v,
                     semA, semB):
    c = core
    s = lax.axis_index("s")
    iota = lax.iota(jnp.int32, 16)
    bn = BIN_SIZE
    sems = (semA, semB)

    def issue_gathers(j16, p):
        cp0 = pltpu.async_copy(f_hbm.at[list_eid.at[pl.ds(j16, 16)]],
                               frows2[p], sems[p])
        cp1 = pltpu.async_copy(so_hbm.at[list_src.at[pl.ds(j16, 16)]],
                               srows2[p], sems[p])
        cp2 = pltpu.async_copy(nv_hbm.at[list_src.at[pl.ds(j16, 16)]],
                               vrows2[p], sems[p])
        return cp0, cp1, cp2

    def bin_body(b, carry_b):
        lo = (c * N_BINS + b) * bn

        @pl.when(s == 0)
        def _init():
            pltpu.sync_copy(ns_hbm.at[pl.ds(lo, bn)], acc_s.at[pl.ds(0, bn)])
            pltpu.sync_copy(nv_hbm.at[pl.ds(lo, bn)], acc_v.at[pl.ds(0, bn)])

        plsc.subcore_barrier()

        def seg_body(g, carry_g):
            seg_base = s * E_PER_TILE + g * SEG
            pltpu.sync_copy(edst_hbm.at[pl.ds(seg_base, SEG)], dseg)
            pltpu.sync_copy(esrc_hbm.at[pl.ds(seg_base, SEG)], sseg)

            def scan_body(i, cnt):
                base = pl.multiple_of(i * 16, 16)
                d = dseg[pl.ds(base, 16)]
                m = (d >= lo) & (d < lo + bn)
                mi = m.astype(jnp.int32)
                pos = cnt + plsc.cumsum(mi) - 1
                plsc.store_scatter(list_eid, [pos], seg_base + base + iota,
                                   mask=m)
                plsc.store_scatter(list_src, [pos], sseg[pl.ds(base, 16)],
                                   mask=m)
                plsc.store_scatter(list_dstl, [pos], d - lo, mask=m)
                return cnt + jnp.sum(mi)

            cnt = lax.fori_loop(0, SEG_VREG, scan_body, jnp.int32(0))

            for extra in (0, 16):
                ppos = cnt + extra + iota
                plsc.store_scatter(list_eid, [ppos],
                                   jnp.zeros((16,), jnp.int32))
                plsc.store_scatter(list_src, [ppos],
                                   jnp.zeros((16,), jnp.int32))
                plsc.store_scatter(list_dstl, [ppos],
                                   jnp.full((16,), bn, jnp.int32))
            nb = lax.shift_right_logical(cnt + 15, 4)

            def compute_and_scatter(j, p):
                idx_dst[...] = list_dstl[pl.ds(pl.multiple_of(j * 16, 16), 16)]
                frows = frows2[p]
                srows = srows2[p]
                vrows = vrows2[p]
                msg_s = msgs2[p]

                def edge_body(k, carry2):
                    kf = jnp.full((16,), k, jnp.int32)
                    zz = jnp.zeros((16,), jnp.int32)
                    d0 = plsc.load_gather(
                        frows, [kf, zz, jnp.full((16,), 768, jnp.int32)])
                    d1 = plsc.load_gather(
                        frows, [kf, zz, jnp.full((16,), 769, jnp.int32)])
                    d2 = plsc.load_gather(
                        frows, [kf, zz, jnp.full((16,), 770, jnp.int32)])
                    for ch in range(NODE_SIZE // 16):
                        o = ch * 16
                        a0 = srows[k, 0, pl.ds(o, 16)]
                        a1 = srows[k, 0, pl.ds(256 + o, 16)]
                        a2 = srows[k, 0, pl.ds(512 + o, 16)]
                        f0 = frows[k, 0, pl.ds(o, 16)]
                        f1 = frows[k, 0, pl.ds(256 + o, 16)]
                        f2 = frows[k, 0, pl.ds(512 + o, 16)]
                        gs = f0 * a0
                        ge = f1 * a1
                        msg_s[k, 0, pl.ds(o, 16)] = f2 * a2
                        vrows[k, 0, pl.ds(o, 16)] = (
                            vrows[k, 0, pl.ds(o, 16)] * gs + d0 * ge)
                        vrows[k, 0, pl.ds(256 + o, 16)] = (
                            vrows[k, 0, pl.ds(256 + o, 16)] * gs + d1 * ge)
                        vrows[k, 0, pl.ds(512 + o, 16)] = (
                            vrows[k, 0, pl.ds(512 + o, 16)] * gs + d2 * ge)
                    return carry2

                lax.fori_loop(0, K, edge_body, jnp.int32(0))
                pltpu.sync_copy(msg_s, acc_s.at[idx_dst], add=True)
                pltpu.sync_copy(vrows, acc_v.at[idx_dst], add=True)

            @pl.when(nb > 0)
            def _prologue():
                issue_gathers(0, 0)

            def pair_body(j2, carry):
                for par in (0, 1):
                    j = j2 * 2 + par

                    @pl.when(j < nb)
                    def _do(j=j, par=par):
                        @pl.when(j + 1 < nb)
                        def _next():
                            issue_gathers(
                                pl.multiple_of((j + 1) * 16, 16), 1 - par)

                        pltpu.make_async_copy(
                            f_hbm.at[list_eid.at[pl.ds(0, 16)]],
                            frows2[par], sems[par]).wait()
                        pltpu.make_async_copy(
                            so_hbm.at[list_src.at[pl.ds(0, 16)]],
                            srows2[par], sems[par]).wait()
                        pltpu.make_async_copy(
                            nv_hbm.at[list_src.at[pl.ds(0, 16)]],
                            vrows2[par], sems[par]).wait()
                        compute_and_scatter(j, par)

                return carry

            lax.fori_loop(0, (nb + 1) // 2, pair_body, jnp.int32(0))
            return carry_g

        lax.fori_loop(0, N_SEG, seg_body, jnp.int32(0))
        plsc.subcore_barrier()

        lo_local = b * bn

        @pl.when(s == 0)
        def _writeout():
            pltpu.sync_copy(acc_s.at[pl.ds(0, bn)],
                            out_s.at[pl.ds(lo_local, bn)])
            pltpu.sync_copy(acc_v.at[pl.ds(0, bn)],
                            out_v.at[pl.ds(lo_local, bn)])

        plsc.subcore_barrier()
        return carry_b

    lax.fori_loop(0, N_BINS, bin_body, jnp.int32(0))


def kernel(node_scalar, node_vector, edge_index, edge_diff, edge_dist,
           W1, b1, W2, b2, Wf, bf):
    scalar_out = _node_mlp(node_scalar, W1, b1, W2, b2)
    f_packed = _edge_filter(edge_dist, edge_diff, Wf, bf)
    so3 = scalar_out.reshape(N_NODES, 1, NODE_SIZE * 3)
    nv3 = node_vector.reshape(N_NODES, 1, NODE_SIZE * 3)
    f3 = f_packed.reshape(N_EDGES, 1, F_WIDTH)
    ns3 = node_scalar.reshape(N_NODES, 1, NODE_SIZE)
    edge_dst = edge_index[0]
    edge_src = edge_index[1]
    s0, v0 = _build_sc_message(0)(so3, nv3, f3, edge_dst, edge_src, ns3)
    s1, v1 = _build_sc_message(1)(so3, nv3, f3, edge_dst, edge_src, ns3)
    out_s = jnp.concatenate([s0, s1], axis=0)
    out_v = jnp.concatenate([v0, v1], axis=0)
    return (out_s.reshape(N_NODES, NODE_SIZE),
            out_v.reshape(N_NODES, 3, NODE_SIZE))


# speedup vs baseline: 1.5588x; 1.5588x over previous
import functools

import jax
import jax.numpy as jnp
from jax import lax
from jax.experimental import pallas as pl
from jax.experimental.pallas import tpu as pltpu
from jax.experimental.pallas import tpu_sc as plsc

EPSILON = 1e-8
CUTOFF = 5.0
NODE_SIZE = 256
EDGE_SIZE = 20
N_NODES = 10000
N_EDGES = 160000

F_WIDTH = 896


_MLP_BLOCK = 1000


def _mlp_body(x_ref, w1_ref, b1_ref, w2_ref, b2_ref, o_ref):
    x = x_ref[...]
    h = jnp.dot(x, w1_ref[...], preferred_element_type=jnp.float32) + b1_ref[...]
    h = h * jax.nn.sigmoid(h)
    o_ref[...] = jnp.dot(h, w2_ref[...], preferred_element_type=jnp.float32) + b2_ref[...]


def _node_mlp(node_scalar, W1, b1, W2, b2):
    n_blocks = N_NODES // _MLP_BLOCK
    return pl.pallas_call(
        _mlp_body,
        grid=(n_blocks,),
        in_specs=[
            pl.BlockSpec((_MLP_BLOCK, NODE_SIZE), lambda i: (i, 0)),
            pl.BlockSpec((NODE_SIZE, NODE_SIZE), lambda i: (0, 0)),
            pl.BlockSpec((1, NODE_SIZE), lambda i: (0, 0)),
            pl.BlockSpec((NODE_SIZE, NODE_SIZE * 3), lambda i: (0, 0)),
            pl.BlockSpec((1, NODE_SIZE * 3), lambda i: (0, 0)),
        ],
        out_specs=pl.BlockSpec((_MLP_BLOCK, NODE_SIZE * 3), lambda i: (i, 0)),
        out_shape=jax.ShapeDtypeStruct((N_NODES, NODE_SIZE * 3), jnp.float32),
    )(node_scalar, W1, b1.reshape(1, -1), W2, b2.reshape(1, -1))


_EDGE_BLOCK = 2000


def _filter_body(dist_ref, diff_ref, wf_ref, bf_ref, o_ref):
    d = dist_ref[...]
    dm = jnp.maximum(d, EPSILON)
    freq = (lax.broadcasted_iota(jnp.int32, (1, EDGE_SIZE), 1).astype(jnp.float32)
            + 1.0) * (jnp.pi / CUTOFF)
    basis = jnp.sin(dm * freq) / dm
    fw = jnp.dot(basis, wf_ref[...], preferred_element_type=jnp.float32) + bf_ref[...]
    cut = jnp.where(d < CUTOFF, 0.5 * (jnp.cos(d * (jnp.pi / CUTOFF)) + 1.0), 0.0)
    fw = fw * cut
    dirpad = jnp.concatenate(
        [diff_ref[...] / dm, jnp.zeros((_EDGE_BLOCK, 125), jnp.float32)], axis=1)
    o_ref[...] = jnp.concatenate([fw, dirpad], axis=1)


def _edge_filter(edge_dist, edge_diff, Wf, bf):
    n_blocks = N_EDGES // _EDGE_BLOCK
    return pl.pallas_call(
        _filter_body,
        grid=(n_blocks,),
        in_specs=[
            pl.BlockSpec((_EDGE_BLOCK, 1), lambda i: (i, 0)),
            pl.BlockSpec((_EDGE_BLOCK, 3), lambda i: (i, 0)),
            pl.BlockSpec((EDGE_SIZE, NODE_SIZE * 3), lambda i: (0, 0)),
            pl.BlockSpec((1, NODE_SIZE * 3), lambda i: (0, 0)),
        ],
        out_specs=pl.BlockSpec((_EDGE_BLOCK, F_WIDTH), lambda i: (i, 0)),
        out_shape=jax.ShapeDtypeStruct((N_EDGES, F_WIDTH), jnp.float32),
    )(edge_dist.reshape(N_EDGES, 1), edge_diff, Wf, bf.reshape(1, -1))


BIN_SIZE = 500
N_BINS = 10
ACC_ROWS = BIN_SIZE + 8
E_PER_TILE = N_EDGES // 16
SEG = 2000
N_SEG = E_PER_TILE // SEG
SEG_VREG = SEG // 16
LIST_LEN = SEG + 32
K = 16


@functools.cache
def _build_sc_message():
    mesh = plsc.VectorSubcoreMesh(core_axis_name="c", subcore_axis_name="s")
    return pl.kernel(
        _sc_message_body,
        mesh=mesh,
        out_type=(
            jax.ShapeDtypeStruct((N_NODES, 1, NODE_SIZE), jnp.float32),
            jax.ShapeDtypeStruct((N_NODES, 1, NODE_SIZE * 3), jnp.float32),
        ),
        scratch_types=[
            pltpu.VMEM((SEG,), jnp.int32),
            pltpu.VMEM((SEG,), jnp.int32),
            pltpu.VMEM((LIST_LEN,), jnp.int32),
            pltpu.VMEM((LIST_LEN,), jnp.int32),
            pltpu.VMEM((LIST_LEN,), jnp.int32),
            pltpu.VMEM((16,), jnp.int32),
            [pltpu.VMEM((K, 1, F_WIDTH), jnp.float32) for _ in range(2)],
            [pltpu.VMEM((K, 1, NODE_SIZE * 3), jnp.float32) for _ in range(2)],
            [pltpu.VMEM((K, 1, NODE_SIZE * 4), jnp.float32) for _ in range(2)],
            pltpu.VMEM_SHARED((ACC_ROWS, 1, NODE_SIZE * 4), jnp.float32),
            pltpu.SemaphoreType.DMA,
            pltpu.SemaphoreType.DMA,
        ],
        compiler_params=pltpu.CompilerParams(needs_layout_passes=False),
    )


def _sc_message_body(so_hbm, nv_hbm, f_hbm, edst_hbm, esrc_hbm, ns_hbm,
                     out_s, out_v,
                     dseg, sseg, list_eid, list_src, list_dstl, idx_dst,
                     frows2, srows2, msg2, acc,
                     semA, semB):
    c = lax.axis_index("c")
    s = lax.axis_index("s")
    iota = lax.iota(jnp.int32, 16)
    bn = BIN_SIZE
    sems = (semA, semB)

    def issue_gathers(j16, p):
        cp0 = pltpu.async_copy(f_hbm.at[list_eid.at[pl.ds(j16, 16)]],
                               frows2[p], sems[p])
        cp1 = pltpu.async_copy(so_hbm.at[list_src.at[pl.ds(j16, 16)]],
                               srows2[p], sems[p])
        cp2 = pltpu.async_copy(
            nv_hbm.at[list_src.at[pl.ds(j16, 16)]],
            msg2[p].at[:, :, pl.ds(NODE_SIZE, NODE_SIZE * 3)], sems[p])
        return cp0, cp1, cp2

    def bin_body(b, carry_b):
        lo = (c * N_BINS + b) * bn

        @pl.when(s == 0)
        def _init():
            pltpu.sync_copy(ns_hbm.at[pl.ds(lo, bn)],
                            acc.at[pl.ds(0, bn), :, pl.ds(0, NODE_SIZE)])
            pltpu.sync_copy(
                nv_hbm.at[pl.ds(lo, bn)],
                acc.at[pl.ds(0, bn), :, pl.ds(NODE_SIZE, NODE_SIZE * 3)])

        plsc.subcore_barrier()

        def seg_body(g, carry_g):
            seg_base = s * E_PER_TILE + g * SEG
            pltpu.sync_copy(edst_hbm.at[pl.ds(seg_base, SEG)], dseg)
            pltpu.sync_copy(esrc_hbm.at[pl.ds(seg_base, SEG)], sseg)

            def scan_body(i, cnt):
                base = pl.multiple_of(i * 16, 16)
                d = dseg[pl.ds(base, 16)]
                m = (d >= lo) & (d < lo + bn)
                mi = m.astype(jnp.int32)
                pos = cnt + plsc.cumsum(mi) - 1
                plsc.store_scatter(list_eid, [pos], seg_base + base + iota,
                                   mask=m)
                plsc.store_scatter(list_src, [pos], sseg[pl.ds(base, 16)],
                                   mask=m)
                plsc.store_scatter(list_dstl, [pos], d - lo, mask=m)
                return cnt + jnp.sum(mi)

            cnt = lax.fori_loop(0, SEG_VREG, scan_body, jnp.int32(0))

            for extra in (0, 16):
                ppos = cnt + extra + iota
                plsc.store_scatter(list_eid, [ppos],
                                   jnp.zeros((16,), jnp.int32))
                plsc.store_scatter(list_src, [ppos],
                                   jnp.zeros((16,), jnp.int32))
                plsc.store_scatter(list_dstl, [ppos],
                                   jnp.full((16,), bn, jnp.int32))
            nb = lax.shift_right_logical(cnt + 15, 4)

            def compute_and_scatter(j, p):
                idx_dst[...] = list_dstl[pl.ds(pl.multiple_of(j * 16, 16), 16)]
                frows = frows2[p]
                srows = srows2[p]
                msg = msg2[p]

                def edge_body(k, carry2):
                    kf = jnp.full((16,), k, jnp.int32)
                    zz = jnp.zeros((16,), jnp.int32)
                    d0 = plsc.load_gather(
                        frows, [kf, zz, jnp.full((16,), 768, jnp.int32)])
                    d1 = plsc.load_gather(
                        frows, [kf, zz, jnp.full((16,), 769, jnp.int32)])
                    d2 = plsc.load_gather(
                        frows, [kf, zz, jnp.full((16,), 770, jnp.int32)])
                    for ch in range(NODE_SIZE // 16):
                        o = ch * 16
                        a0 = srows[k, 0, pl.ds(o, 16)]
                        a1 = srows[k, 0, pl.ds(256 + o, 16)]
                        a2 = srows[k, 0, pl.ds(512 + o, 16)]
                        f0 = frows[k, 0, pl.ds(o, 16)]
                        f1 = frows[k, 0, pl.ds(256 + o, 16)]
                        f2 = frows[k, 0, pl.ds(512 + o, 16)]
                        gs = f0 * a0
                        ge = f1 * a1
                        msg[k, 0, pl.ds(o, 16)] = f2 * a2
                        msg[k, 0, pl.ds(256 + o, 16)] = (
                            msg[k, 0, pl.ds(256 + o, 16)] * gs + d0 * ge)
                        msg[k, 0, pl.ds(512 + o, 16)] = (
                            msg[k, 0, pl.ds(512 + o, 16)] * gs + d1 * ge)
                        msg[k, 0, pl.ds(768 + o, 16)] = (
                            msg[k, 0, pl.ds(768 + o, 16)] * gs + d2 * ge)
                    return carry2

                lax.fori_loop(0, K, edge_body, jnp.int32(0))
                pltpu.sync_copy(msg, acc.at[idx_dst], add=True)

            @pl.when(nb > 0)
            def _prologue():
                issue_gathers(0, 0)

            def pair_body(j2, carry):
                for par in (0, 1):
                    j = j2 * 2 + par

                    @pl.when(j < nb)
                    def _do(j=j, par=par):
                        @pl.when(j + 1 < nb)
                        def _next():
                            issue_gathers(
                                pl.multiple_of((j + 1) * 16, 16), 1 - par)

                        pltpu.make_async_copy(
                            f_hbm.at[list_eid.at[pl.ds(0, 16)]],
                            frows2[par], sems[par]).wait()
                        pltpu.make_async_copy(
                            so_hbm.at[list_src.at[pl.ds(0, 16)]],
                            srows2[par], sems[par]).wait()
                        pltpu.make_async_copy(
                            nv_hbm.at[list_src.at[pl.ds(0, 16)]],
                            msg2[par].at[:, :, pl.ds(NODE_SIZE, NODE_SIZE * 3)],
                            sems[par]).wait()
                        compute_and_scatter(j, par)

                return carry

            lax.fori_loop(0, (nb + 1) // 2, pair_body, jnp.int32(0))
            return carry_g

        lax.fori_loop(0, N_SEG, seg_body, jnp.int32(0))
        plsc.subcore_barrier()

        @pl.when(s == 0)
        def _writeout():
            pltpu.sync_copy(acc.at[pl.ds(0, bn), :, pl.ds(0, NODE_SIZE)],
                            out_s.at[pl.ds(lo, bn)])
            pltpu.sync_copy(
                acc.at[pl.ds(0, bn), :, pl.ds(NODE_SIZE, NODE_SIZE * 3)],
                out_v.at[pl.ds(lo, bn)])

        plsc.subcore_barrier()
        return carry_b

    lax.fori_loop(0, N_BINS, bin_body, jnp.int32(0))


def kernel(node_scalar, node_vector, edge_index, edge_diff, edge_dist,
           W1, b1, W2, b2, Wf, bf):
    scalar_out = _node_mlp(node_scalar, W1, b1, W2, b2)
    f_packed = _edge_filter(edge_dist, edge_diff, Wf, bf)
    so3 = scalar_out.reshape(N_NODES, 1, NODE_SIZE * 3)
    nv3 = node_vector.reshape(N_NODES, 1, NODE_SIZE * 3)
    f3 = f_packed.reshape(N_EDGES, 1, F_WIDTH)
    ns3 = node_scalar.reshape(N_NODES, 1, NODE_SIZE)
    edge_dst = edge_index[0]
    edge_src = edge_index[1]
    out_s, out_v = _build_sc_message()(so3, nv3, f3, edge_dst, edge_src, ns3)
    return (out_s.reshape(N_NODES, NODE_SIZE),
            out_v.reshape(N_NODES, 3, NODE_SIZE))


# speedup vs baseline: 1.5658x; 1.0045x over previous
import functools

import jax
import jax.numpy as jnp
from jax import lax
from jax.experimental import pallas as pl
from jax.experimental.pallas import tpu as pltpu
from jax.experimental.pallas import tpu_sc as plsc

EPSILON = 1e-8
CUTOFF = 5.0
NODE_SIZE = 256
EDGE_SIZE = 20
N_NODES = 10000
N_EDGES = 160000

F_WIDTH = 896


_MLP_BLOCK = 1000


def _mlp_body(x_ref, w1_ref, b1_ref, w2_ref, b2_ref, o_ref):
    x = x_ref[...]
    h = jnp.dot(x, w1_ref[...], preferred_element_type=jnp.float32) + b1_ref[...]
    h = h * jax.nn.sigmoid(h)
    o_ref[...] = jnp.dot(h, w2_ref[...], preferred_element_type=jnp.float32) + b2_ref[...]


def _node_mlp(node_scalar, W1, b1, W2, b2):
    n_blocks = N_NODES // _MLP_BLOCK
    return pl.pallas_call(
        _mlp_body,
        grid=(n_blocks,),
        in_specs=[
            pl.BlockSpec((_MLP_BLOCK, NODE_SIZE), lambda i: (i, 0)),
            pl.BlockSpec((NODE_SIZE, NODE_SIZE), lambda i: (0, 0)),
            pl.BlockSpec((1, NODE_SIZE), lambda i: (0, 0)),
            pl.BlockSpec((NODE_SIZE, NODE_SIZE * 3), lambda i: (0, 0)),
            pl.BlockSpec((1, NODE_SIZE * 3), lambda i: (0, 0)),
        ],
        out_specs=pl.BlockSpec((_MLP_BLOCK, NODE_SIZE * 3), lambda i: (i, 0)),
        out_shape=jax.ShapeDtypeStruct((N_NODES, NODE_SIZE * 3), jnp.float32),
    )(node_scalar, W1, b1.reshape(1, -1), W2, b2.reshape(1, -1))


_EDGE_BLOCK = 2000


def _filter_body(dist_ref, diff_ref, wf_ref, bf_ref, o_ref):
    d = dist_ref[...]
    dm = jnp.maximum(d, EPSILON)
    freq = (lax.broadcasted_iota(jnp.int32, (1, EDGE_SIZE), 1).astype(jnp.float32)
            + 1.0) * (jnp.pi / CUTOFF)
    basis = jnp.sin(dm * freq) / dm
    fw = jnp.dot(basis, wf_ref[...], preferred_element_type=jnp.float32) + bf_ref[...]
    cut = jnp.where(d < CUTOFF, 0.5 * (jnp.cos(d * (jnp.pi / CUTOFF)) + 1.0), 0.0)
    fw = fw * cut
    dirpad = jnp.concatenate(
        [diff_ref[...] / dm, jnp.zeros((_EDGE_BLOCK, 125), jnp.float32)], axis=1)
    o_ref[...] = jnp.concatenate([fw, dirpad], axis=1)


def _edge_filter(edge_dist, edge_diff, Wf, bf):
    n_blocks = N_EDGES // _EDGE_BLOCK
    return pl.pallas_call(
        _filter_body,
        grid=(n_blocks,),
        in_specs=[
            pl.BlockSpec((_EDGE_BLOCK, 1), lambda i: (i, 0)),
            pl.BlockSpec((_EDGE_BLOCK, 3), lambda i: (i, 0)),
            pl.BlockSpec((EDGE_SIZE, NODE_SIZE * 3), lambda i: (0, 0)),
            pl.BlockSpec((1, NODE_SIZE * 3), lambda i: (0, 0)),
        ],
        out_specs=pl.BlockSpec((_EDGE_BLOCK, F_WIDTH), lambda i: (i, 0)),
        out_shape=jax.ShapeDtypeStruct((N_EDGES, F_WIDTH), jnp.float32),
    )(edge_dist.reshape(N_EDGES, 1), edge_diff, Wf, bf.reshape(1, -1))


BIN_SIZE = 500
N_BINS = 10
ACC_ROWS = BIN_SIZE + 8
E_PER_TILE = N_EDGES // 16
SEG = 2000
N_SEG = E_PER_TILE // SEG
SEG_VREG = SEG // 16
LIST_LEN = SEG + 32
K = 16


@functools.cache
def _build_sc_message():
    mesh = plsc.VectorSubcoreMesh(core_axis_name="c", subcore_axis_name="s")
    return pl.kernel(
        _sc_message_body,
        mesh=mesh,
        out_type=(
            jax.ShapeDtypeStruct((N_NODES, 1, NODE_SIZE), jnp.float32),
            jax.ShapeDtypeStruct((N_NODES, 1, NODE_SIZE * 3), jnp.float32),
        ),
        scratch_types=[
            pltpu.VMEM((SEG,), jnp.int32),
            pltpu.VMEM((SEG,), jnp.int32),
            pltpu.VMEM((LIST_LEN,), jnp.int32),
            pltpu.VMEM((LIST_LEN,), jnp.int32),
            pltpu.VMEM((LIST_LEN,), jnp.int32),
            [pltpu.VMEM((16,), jnp.int32) for _ in range(2)],
            [pltpu.VMEM((K, 1, F_WIDTH), jnp.float32) for _ in range(2)],
            [pltpu.VMEM((K, 1, NODE_SIZE * 3), jnp.float32) for _ in range(2)],
            [pltpu.VMEM((K, 1, NODE_SIZE * 4), jnp.float32) for _ in range(2)],
            pltpu.VMEM_SHARED((ACC_ROWS, 1, NODE_SIZE * 4), jnp.float32),
            pltpu.SemaphoreType.DMA,
            pltpu.SemaphoreType.DMA,
            pltpu.SemaphoreType.DMA,
            pltpu.SemaphoreType.DMA,
        ],
        compiler_params=pltpu.CompilerParams(needs_layout_passes=False),
    )


def _sc_message_body(so_hbm, nv_hbm, f_hbm, edst_hbm, esrc_hbm, ns_hbm,
                     out_s, out_v,
                     dseg, sseg, list_eid, list_src, list_dstl, idx_dst2,
                     frows2, srows2, msg2, acc,
                     semA, semB, semC, semD):
    c = lax.axis_index("c")
    s = lax.axis_index("s")
    iota = lax.iota(jnp.int32, 16)
    bn = BIN_SIZE
    sems = (semA, semB)
    sems_sc = (semC, semD)

    def issue_gathers(j16, p):
        cp0 = pltpu.async_copy(f_hbm.at[list_eid.at[pl.ds(j16, 16)]],
                               frows2[p], sems[p])
        cp1 = pltpu.async_copy(so_hbm.at[list_src.at[pl.ds(j16, 16)]],
                               srows2[p], sems[p])
        cp2 = pltpu.async_copy(
            nv_hbm.at[list_src.at[pl.ds(j16, 16)]],
            msg2[p].at[:, :, pl.ds(NODE_SIZE, NODE_SIZE * 3)], sems[p])
        return cp0, cp1, cp2

    def bin_body(b, carry_b):
        lo = (c * N_BINS + b) * bn

        @pl.when(s == 0)
        def _init():
            pltpu.sync_copy(ns_hbm.at[pl.ds(lo, bn)],
                            acc.at[pl.ds(0, bn), :, pl.ds(0, NODE_SIZE)])
            pltpu.sync_copy(
                nv_hbm.at[pl.ds(lo, bn)],
                acc.at[pl.ds(0, bn), :, pl.ds(NODE_SIZE, NODE_SIZE * 3)])

        plsc.subcore_barrier()

        def seg_body(g, carry_g):
            seg_base = s * E_PER_TILE + g * SEG
            pltpu.sync_copy(edst_hbm.at[pl.ds(seg_base, SEG)], dseg)
            pltpu.sync_copy(esrc_hbm.at[pl.ds(seg_base, SEG)], sseg)

            def scan_body(i, cnt):
                base = pl.multiple_of(i * 16, 16)
                d = dseg[pl.ds(base, 16)]
                m = (d >= lo) & (d < lo + bn)
                mi = m.astype(jnp.int32)
                pos = cnt + plsc.cumsum(mi) - 1
                plsc.store_scatter(list_eid, [pos], seg_base + base + iota,
                                   mask=m)
                plsc.store_scatter(list_src, [pos], sseg[pl.ds(base, 16)],
                                   mask=m)
                plsc.store_scatter(list_dstl, [pos], d - lo, mask=m)
                return cnt + jnp.sum(mi)

            cnt = lax.fori_loop(0, SEG_VREG, scan_body, jnp.int32(0))

            for extra in (0, 16):
                ppos = cnt + extra + iota
                plsc.store_scatter(list_eid, [ppos],
                                   jnp.zeros((16,), jnp.int32))
                plsc.store_scatter(list_src, [ppos],
                                   jnp.zeros((16,), jnp.int32))
                plsc.store_scatter(list_dstl, [ppos],
                                   jnp.full((16,), bn, jnp.int32))
            nb = lax.shift_right_logical(cnt + 15, 4)

            def compute_and_scatter(j, p):
                idx_dst = idx_dst2[p]
                idx_dst[...] = list_dstl[pl.ds(pl.multiple_of(j * 16, 16), 16)]
                frows = frows2[p]
                srows = srows2[p]
                msg = msg2[p]

                def edge_body(k, carry2):
                    kf = jnp.full((16,), k, jnp.int32)
                    zz = jnp.zeros((16,), jnp.int32)
                    d0 = plsc.load_gather(
                        frows, [kf, zz, jnp.full((16,), 768, jnp.int32)])
                    d1 = plsc.load_gather(
                        frows, [kf, zz, jnp.full((16,), 769, jnp.int32)])
                    d2 = plsc.load_gather(
                        frows, [kf, zz, jnp.full((16,), 770, jnp.int32)])
                    for ch in range(NODE_SIZE // 16):
                        o = ch * 16
                        a0 = srows[k, 0, pl.ds(o, 16)]
                        a1 = srows[k, 0, pl.ds(256 + o, 16)]
                        a2 = srows[k, 0, pl.ds(512 + o, 16)]
                        f0 = frows[k, 0, pl.ds(o, 16)]
                        f1 = frows[k, 0, pl.ds(256 + o, 16)]
                        f2 = frows[k, 0, pl.ds(512 + o, 16)]
                        gs = f0 * a0
                        ge = f1 * a1
                        msg[k, 0, pl.ds(o, 16)] = f2 * a2
                        msg[k, 0, pl.ds(256 + o, 16)] = (
                            msg[k, 0, pl.ds(256 + o, 16)] * gs + d0 * ge)
                        msg[k, 0, pl.ds(512 + o, 16)] = (
                            msg[k, 0, pl.ds(512 + o, 16)] * gs + d1 * ge)
                        msg[k, 0, pl.ds(768 + o, 16)] = (
                            msg[k, 0, pl.ds(768 + o, 16)] * gs + d2 * ge)
                    return carry2

                lax.fori_loop(0, K, edge_body, jnp.int32(0))
                pltpu.async_copy(msg, acc.at[idx_dst], sems_sc[p], add=True)

            @pl.when(nb > 0)
            def _prologue():
                issue_gathers(0, 0)

            def pair_body(j2, carry):
                for par in (0, 1):
                    j = j2 * 2 + par

                    @pl.when(j < nb)
                    def _do(j=j, par=par):
                        @pl.when(j + 1 < nb)
                        def _next():
                            @pl.when(j >= 1)
                            def _drain_prev():
                                pltpu.make_async_copy(
                                    msg2[1 - par],
                                    acc.at[idx_dst2[1 - par]],
                                    sems_sc[1 - par]).wait()

                            issue_gathers(
                                pl.multiple_of((j + 1) * 16, 16), 1 - par)

                        pltpu.make_async_copy(
                            f_hbm.at[list_eid.at[pl.ds(0, 16)]],
                            frows2[par], sems[par]).wait()
                        pltpu.make_async_copy(
                            so_hbm.at[list_src.at[pl.ds(0, 16)]],
                            srows2[par], sems[par]).wait()
                        pltpu.make_async_copy(
                            nv_hbm.at[list_src.at[pl.ds(0, 16)]],
                            msg2[par].at[:, :, pl.ds(NODE_SIZE, NODE_SIZE * 3)],
                            sems[par]).wait()
                        compute_and_scatter(j, par)

                return carry

            lax.fori_loop(0, (nb + 1) // 2, pair_body, jnp.int32(0))

            @pl.when(nb >= 2)
            def _drain_tail2():
                pltpu.make_async_copy(
                    msg2[0], acc.at[idx_dst2[0]], sems_sc[0]).wait()
                pltpu.make_async_copy(
                    msg2[1], acc.at[idx_dst2[1]], sems_sc[1]).wait()

            @pl.when(nb == 1)
            def _drain_tail1():
                pltpu.make_async_copy(
                    msg2[0], acc.at[idx_dst2[0]], sems_sc[0]).wait()

            return carry_g

        lax.fori_loop(0, N_SEG, seg_body, jnp.int32(0))
        plsc.subcore_barrier()

        @pl.when(s == 0)
        def _writeout():
            pltpu.sync_copy(acc.at[pl.ds(0, bn), :, pl.ds(0, NODE_SIZE)],
                            out_s.at[pl.ds(lo, bn)])
            pltpu.sync_copy(
                acc.at[pl.ds(0, bn), :, pl.ds(NODE_SIZE, NODE_SIZE * 3)],
                out_v.at[pl.ds(lo, bn)])

        plsc.subcore_barrier()
        return carry_b

    lax.fori_loop(0, N_BINS, bin_body, jnp.int32(0))


def kernel(node_scalar, node_vector, edge_index, edge_diff, edge_dist,
           W1, b1, W2, b2, Wf, bf):
    scalar_out = _node_mlp(node_scalar, W1, b1, W2, b2)
    f_packed = _edge_filter(edge_dist, edge_diff, Wf, bf)
    so3 = scalar_out.reshape(N_NODES, 1, NODE_SIZE * 3)
    nv3 = node_vector.reshape(N_NODES, 1, NODE_SIZE * 3)
    f3 = f_packed.reshape(N_EDGES, 1, F_WIDTH)
    ns3 = node_scalar.reshape(N_NODES, 1, NODE_SIZE)
    edge_dst = edge_index[0]
    edge_src = edge_index[1]
    out_s, out_v = _build_sc_message()(so3, nv3, f3, edge_dst, edge_src, ns3)
    return (out_s.reshape(N_NODES, NODE_SIZE),
            out_v.reshape(N_NODES, 3, NODE_SIZE))
